# baseline (speedup 1.0000x reference)
import functools

import jax
import jax.numpy as jnp
from jax import lax
from jax.experimental import pallas as pl
from jax.experimental.pallas import tpu as pltpu
from jax.experimental.pallas import tpu_sc as plsc

_B, _C, _T, _H, _W = 8, 3, 32, 224, 224
_NG = _B * _C
_NSLAB = _NG * _T
_NC, _NS = 2, 16

_PERM = (31, 7, 4, 29, 16, 19, 2, 5, 30, 3, 22, 6, 18, 10, 11, 15,
         20, 8, 24, 9, 25, 13, 14, 17, 23, 0, 21, 26, 1, 28, 27, 12)

_NSEM = 4


@functools.partial(
    pl.kernel,
    out_type=jax.ShapeDtypeStruct((_NSLAB, _H, _W), jnp.float32),
    mesh=plsc.VectorSubcoreMesh(core_axis_name="c", subcore_axis_name="s"),
    scratch_types=[pltpu.SemaphoreType.DMA for _ in range(_NSEM)],
)
def _sc_permute(frames_hbm, out_hbm, *sems):
    wid = lax.axis_index("s") * _NC + lax.axis_index("c")
    src_t = jnp.int32(0)
    for k in range(_T):
        src_t = src_t + jnp.int32(_PERM[k]) * (wid == k).astype(jnp.int32)

    copies = []
    for i in range(_NG):
        copies.append(pltpu.async_copy(
            frames_hbm.at[pl.ds(i * _T + src_t, 1)],
            out_hbm.at[pl.ds(i * _T + wid, 1)],
            sems[i % _NSEM]))
    for c in copies:
        c.wait()


def kernel(frames):
    flat = frames.reshape(_NSLAB, _H, _W)
    out = _sc_permute(flat)
    return out.reshape(frames.shape)


# speedup vs baseline: 37.8079x; 37.8079x over previous
import functools

import jax
import jax.numpy as jnp
import numpy as np
from jax import lax
from jax.experimental import pallas as pl
from jax.experimental.pallas import tpu as pltpu
from jax.experimental.pallas import tpu_sc as plsc

_B, _C, _T, _H, _W = 8, 3, 32, 224, 224
_NG = _B * _C
_NSLAB = _NG * _T
_NC, _NS = 2, 16
_NBUF = 2

_PERM = (31, 7, 4, 29, 16, 19, 2, 5, 30, 3, 22, 6, 18, 10, 11, 15,
         20, 8, 24, 9, 25, 13, 14, 17, 23, 0, 21, 26, 1, 28, 27, 12)


@functools.partial(
    pl.kernel,
    out_type=jax.ShapeDtypeStruct((_NSLAB, _H, _W), jnp.float32),
    mesh=plsc.VectorSubcoreMesh(core_axis_name="c", subcore_axis_name="s"),
    scratch_types=[pltpu.VMEM((1, _H, _W), jnp.float32) for _ in range(_NBUF)]
                  + [pltpu.SemaphoreType.DMA for _ in range(2 * _NBUF)],
)
def _sc_permute(frames_hbm, out_hbm, *rest):
    bufs = rest[:_NBUF]
    gsems = rest[_NBUF:2 * _NBUF]
    osems = rest[2 * _NBUF:]

    wid = lax.axis_index("s") * _NC + lax.axis_index("c")
    src_t = jnp.int32(0)
    for k in range(_T):
        src_t = src_t + jnp.int32(_PERM[k]) * (wid == k).astype(jnp.int32)

    def group(i):
        return lax.rem(jnp.int32(i) + wid, jnp.int32(_NG))

    def gather(i, s):
        return pltpu.async_copy(
            frames_hbm.at[pl.ds(group(i) * _T + src_t, 1)], bufs[s], gsems[s])

    def put(i, s):
        return pltpu.async_copy(
            bufs[s], out_hbm.at[pl.ds(group(i) * _T + wid, 1)], osems[s])

    gathers = [gather(b, b) for b in range(_NBUF)]
    outs = [None] * _NBUF
    for i in range(_NG):
        s = i % _NBUF
        j = i + _NBUF - 1
        if i >= 1 and j < _NG:
            ps = (s - 1) % _NBUF
            outs[ps].wait()
            gathers[ps] = gather(j, ps)
        gathers[s].wait()
        outs[s] = put(i, s)
    for b in range(_NBUF):
        if outs[b] is not None:
            outs[b].wait()


def kernel(frames):
    flat = frames.reshape(_NSLAB, _H, _W)
    out = _sc_permute(flat)
    return out.reshape(frames.shape)


# speedup vs baseline: 38.2534x; 1.0118x over previous
import functools

import jax
import jax.numpy as jnp
from jax import lax
from jax.experimental import pallas as pl
from jax.experimental.pallas import tpu as pltpu
from jax.experimental.pallas import tpu_sc as plsc

_B, _C, _T, _H, _W = 8, 3, 32, 224, 224
_NG = _B * _C
_NSLAB = _NG * _T
_NC, _NS = 2, 16
_NBUF = 2

_PERM = (31, 7, 4, 29, 16, 19, 2, 5, 30, 3, 22, 6, 18, 10, 11, 15,
         20, 8, 24, 9, 25, 13, 14, 17, 23, 0, 21, 26, 1, 28, 27, 12)


@functools.partial(
    pl.kernel,
    out_type=jax.ShapeDtypeStruct((_NSLAB, _H, _W), jnp.float32),
    mesh=plsc.VectorSubcoreMesh(core_axis_name="c", subcore_axis_name="s"),
    scratch_types=[pltpu.VMEM((1, _H, _W), jnp.float32) for _ in range(_NBUF)]
                  + [pltpu.SemaphoreType.DMA for _ in range(2 * _NBUF)],
)
def _sc_permute(frames_hbm, out_hbm, *rest):
    bufs = rest[:_NBUF]
    gsems = rest[_NBUF:2 * _NBUF]
    osems = rest[2 * _NBUF:]

    wid = lax.axis_index("s") * _NC + lax.axis_index("c")
    src_t = jnp.int32(0)
    for k in range(_T):
        src_t = src_t + jnp.int32(_PERM[k]) * (wid == k).astype(jnp.int32)

    def gather(i, s):
        return pltpu.async_copy(
            frames_hbm.at[pl.ds(i * _T + src_t, 1)], bufs[s], gsems[s])

    def put(i, s):
        return pltpu.async_copy(
            bufs[s], out_hbm.at[pl.ds(i * _T + wid, 1)], osems[s])

    gathers = [gather(b, b) for b in range(_NBUF)]
    outs = [None] * _NBUF
    for i in range(_NG):
        s = i % _NBUF
        j = i + _NBUF - 1
        if i >= 1 and j < _NG:
            ps = (s - 1) % _NBUF
            outs[ps].wait()
            gathers[ps] = gather(j, ps)
        gathers[s].wait()
        outs[s] = put(i, s)
    for b in range(_NBUF):
        if outs[b] is not None:
            outs[b].wait()


def kernel(frames):
    flat = frames.reshape(_NSLAB, _H, _W)
    out = _sc_permute(flat)
    return out.reshape(frames.shape)
